# baseline (speedup 1.0000x reference)
import jax
import jax.numpy as jnp
from jax.experimental import pallas as pl

_K = 8192
_KT = 2048
_P = 1024
_L = 128


def _labeller_kernel(x_ref, c_ref, out_ref):
    x = x_ref[0]
    xm2 = -2.0 * x
    x2 = jnp.sum(x * x, axis=1, keepdims=True)

    lane = jax.lax.broadcasted_iota(jnp.int32, (_P, _L), 1)
    best = jnp.full((_P, _L), jnp.inf, dtype=jnp.float32)
    bestc = jnp.zeros((_P, _L), dtype=jnp.int32)
    for t in range(_K // _KT):
        ct = c_ref[t * _KT:(t + 1) * _KT, :]
        c2 = jnp.sum(ct * ct, axis=1)[None, :]
        dm2 = jax.lax.dot_general(
            xm2, ct, (((1,), (1,)), ((), ())),
            preferred_element_type=jnp.float32)
        for c in range(_KT // _L):
            sl = slice(c * _L, (c + 1) * _L)
            s = (x2 + c2[:, sl]) + dm2[:, sl]
            cid = t * (_KT // _L) + c
            upd = s < best
            best = jnp.where(upd, s, best)
            bestc = jnp.where(upd, jnp.full((_P, _L), cid, jnp.int32), bestc)

    idx = bestc * _L + lane
    m = jnp.min(best, axis=1, keepdims=True)
    out_ref[0, 0, :] = jnp.min(jnp.where(best == m, idx, _K), axis=1)


def kernel(inpt, cluster_centers):
    b, p, d = inpt.shape
    labels = pl.pallas_call(
        _labeller_kernel,
        grid=(b,),
        in_specs=[
            pl.BlockSpec((1, p, d), lambda i: (i, 0, 0)),
            pl.BlockSpec(cluster_centers.shape, lambda i: (0, 0)),
        ],
        out_specs=pl.BlockSpec((1, 1, p), lambda i: (i, 0, 0)),
        out_shape=jax.ShapeDtypeStruct((b, 1, p), jnp.int32),
    )(inpt, cluster_centers)
    return labels.reshape(b, p)


# speedup vs baseline: 1.2103x; 1.2103x over previous
import jax
import jax.numpy as jnp
from jax.experimental import pallas as pl
from jax.experimental.pallas import tpu as pltpu

_K = 8192
_KT = 2048
_PB = 1024
_PS = 64
_L = 128


def _labeller_kernel(x_ref, ct_ref, out_ref, c2_ref):
    @pl.when(pl.program_id(0) == 0)
    def _():
        cc = ct_ref[...]
        c2_ref[...] = jnp.sum(cc * cc, axis=0, keepdims=True)

    x = x_ref[...]
    xm2 = -2.0 * x
    x2 = jnp.sum(x * x, axis=1, keepdims=True)
    x2b = jnp.broadcast_to(x2, (_PB, _L))
    c2 = c2_ref[...]

    nps = _PB // _PS
    ncs = _KT // _L
    bests = [jnp.full((_PS, _L), jnp.inf, dtype=jnp.float32)] * nps
    bestcs = [jnp.zeros((_PS, _L), dtype=jnp.int32)] * nps
    for t in range(_K // _KT):
        dm2 = jax.lax.dot_general(
            xm2, ct_ref[:, t * _KT:(t + 1) * _KT], (((1,), (0,)), ((), ())),
            preferred_element_type=jnp.float32)
        for pb in range(nps):
            rs = slice(pb * _PS, (pb + 1) * _PS)
            x2s = x2b[rs]
            best, bestc = bests[pb], bestcs[pb]
            for c in range(ncs):
                cid = t * ncs + c
                sl = slice(cid * _L, (cid + 1) * _L)
                s = (x2s + c2[:, sl]) + dm2[rs, c * _L:(c + 1) * _L]
                upd = s < best
                best = jnp.where(upd, s, best)
                bestc = jnp.where(upd, jnp.full((_PS, _L), cid, jnp.int32), bestc)
            bests[pb], bestcs[pb] = best, bestc

    lane = jax.lax.broadcasted_iota(jnp.int32, (_PS, _L), 1)
    for pb in range(nps):
        rs = slice(pb * _PS, (pb + 1) * _PS)
        best, bestc = bests[pb], bestcs[pb]
        idx = bestc * _L + lane
        m = jnp.min(best, axis=1, keepdims=True)
        out_ref[0, 0, rs] = jnp.min(jnp.where(best == m, idx, _K), axis=1)


def kernel(inpt, cluster_centers):
    b, p, d = inpt.shape
    n = b * p
    x = inpt.reshape(n, d)
    ct = cluster_centers.T
    labels = pl.pallas_call(
        _labeller_kernel,
        grid=(n // _PB,),
        in_specs=[
            pl.BlockSpec((_PB, d), lambda i: (i, 0)),
            pl.BlockSpec(ct.shape, lambda i: (0, 0)),
        ],
        out_specs=pl.BlockSpec((1, 1, _PB), lambda i: (i, 0, 0)),
        out_shape=jax.ShapeDtypeStruct((n // _PB, 1, _PB), jnp.int32),
        scratch_shapes=[pltpu.VMEM((1, _K), jnp.float32)],
    )(x, ct)
    return labels.reshape(b, p)
